# baseline (speedup 1.0000x reference)
import functools

import jax
import jax.numpy as jnp
from jax import lax
from jax.experimental import pallas as pl
from jax.experimental.pallas import tpu as pltpu
from jax.experimental.pallas import tpu_sc as plsc

D = 32
B = 16384
L = 50
NC = 2
NS = 16
NW = NC * NS
BW = B // NW
CL = 128
CH = BW // CL

_mesh = plsc.VectorSubcoreMesh(core_axis_name="c", subcore_axis_name="s")


@functools.partial(
    pl.kernel,
    mesh=_mesh,
    out_type=jax.ShapeDtypeStruct((B,), jnp.float32),
    compiler_params=pltpu.CompilerParams(
        needs_layout_passes=False, use_tc_tiling_on_sc=False),
    scratch_types=[
        pltpu.VMEM((L, CH, CL), jnp.int32),
        pltpu.VMEM((CH, CL), jnp.int32),
        pltpu.VMEM((CH, CL), jnp.int32),
        pltpu.VMEM((BW,), jnp.float32),
        pltpu.VMEM((BW, D), jnp.float32),
        pltpu.VMEM((BW, D), jnp.float32),
        pltpu.VMEM((BW, D), jnp.float32),
        pltpu.VMEM((BW, D), jnp.float32),
        pltpu.VMEM((CL, D), jnp.float32),
        pltpu.VMEM((BW,), jnp.float32),
        pltpu.SemaphoreType.DMA,
    ],
)
def _fpmc_sc(seq_idx_hbm, user_hbm, item_hbm, seqlen_hbm,
             wui_hbm, wiu_hbm, wli_hbm, wil_hbm, out_hbm,
             seq_idx_v, user_v, item_v, seqlen_v,
             vui_v, viu_v, vil_v, acc_v, buf_v, out_v, sem0):
    wid = lax.axis_index("s") * NC + lax.axis_index("c")
    base = wid * BW

    pltpu.sync_copy(seq_idx_hbm.at[wid], seq_idx_v)
    pltpu.sync_copy(user_hbm.at[wid], user_v)
    pltpu.sync_copy(item_hbm.at[wid], item_v)
    pltpu.sync_copy(seqlen_hbm.at[wid], seqlen_v)

    for c in range(CH):
        dst = pl.ds(c * CL, CL)
        pltpu.async_copy(wui_hbm.at[user_v.at[c]], vui_v.at[dst], sem0).wait()
        pltpu.async_copy(wiu_hbm.at[item_v.at[c]], viu_v.at[dst], sem0).wait()
        pltpu.async_copy(wil_hbm.at[item_v.at[c]], vil_v.at[dst], sem0).wait()

    zero = jnp.zeros((16,), jnp.float32)

    def zbody(i, carry):
        acc_v[i, pl.ds(0, 16)] = zero
        acc_v[i, pl.ds(16, 16)] = zero
        return carry
    lax.fori_loop(0, BW, zbody, 0)

    def lbody(l, carry):
        def cbody(c, carry2):
            pltpu.async_copy(wli_hbm.at[seq_idx_v.at[l, c]], buf_v, sem0).wait()

            def abody(i, carry3):
                r = c * CL + i
                plsc.addupdate(acc_v.at[r, pl.ds(0, 16)], buf_v[i, pl.ds(0, 16)])
                plsc.addupdate(acc_v.at[r, pl.ds(16, 16)], buf_v[i, pl.ds(16, 16)])
                return carry3
            lax.fori_loop(0, CL, abody, 0)
            return carry2
        lax.fori_loop(0, CH, cbody, 0)
        return carry
    lax.fori_loop(0, L, lbody, 0)

    iota16 = lax.iota(jnp.int32, 16)

    def gbody(g, carry):
        rows = g * 16 + iota16
        a1 = zero
        a2 = zero
        for d in range(D):
            cols = jnp.full((16,), d, jnp.int32)
            vu = plsc.load_gather(vui_v, [rows, cols])
            vi = plsc.load_gather(viu_v, [rows, cols])
            sacc = plsc.load_gather(acc_v, [rows, cols])
            vl = plsc.load_gather(vil_v, [rows, cols])
            a1 = a1 + vu * vi
            a2 = a2 + sacc * vl
        off = pl.multiple_of(g * 16, 16)
        sl = seqlen_v[pl.ds(off, 16)]
        out_v[pl.ds(off, 16)] = a1 + a2 / sl
        return carry
    lax.fori_loop(0, BW // 16, gbody, 0)

    pltpu.sync_copy(out_v, out_hbm.at[pl.ds(base, BW)])


def kernel(user, item, item_seq, seq_len, W_UI, W_IU, W_LI, W_IL):
    user_r = user.astype(jnp.int32).reshape(NW, CH, CL)
    item_r = item.astype(jnp.int32).reshape(NW, CH, CL)
    seq_r = item_seq.astype(jnp.int32).reshape(NW, CH, CL, L).transpose(0, 3, 1, 2)
    sl_r = seq_len.reshape(NW, BW)
    return _fpmc_sc(seq_r, user_r, item_r, sl_r, W_UI, W_IU, W_LI, W_IL)


# speedup vs baseline: 1.1239x; 1.1239x over previous
import functools

import jax
import jax.numpy as jnp
from jax import lax
from jax.experimental import pallas as pl
from jax.experimental.pallas import tpu as pltpu
from jax.experimental.pallas import tpu_sc as plsc

D = 32
B = 16384
L = 50
NC = 2
NS = 16
NW = NC * NS
BW = B // NW
CL = 128
CH = BW // CL

_mesh = plsc.VectorSubcoreMesh(core_axis_name="c", subcore_axis_name="s")


@functools.partial(
    pl.kernel,
    mesh=_mesh,
    out_type=jax.ShapeDtypeStruct((B,), jnp.float32),
    compiler_params=pltpu.CompilerParams(
        needs_layout_passes=False, use_tc_tiling_on_sc=False),
    scratch_types=[
        pltpu.VMEM((L, CH, CL), jnp.int32),
        pltpu.VMEM((CH, CL), jnp.int32),
        pltpu.VMEM((CH, CL), jnp.int32),
        pltpu.VMEM((BW,), jnp.float32),
        pltpu.VMEM((BW, D), jnp.float32),
        pltpu.VMEM((BW, D), jnp.float32),
        pltpu.VMEM((BW, D), jnp.float32),
        pltpu.VMEM((BW, D), jnp.float32),
        pltpu.VMEM((CL, D), jnp.float32),
        pltpu.VMEM((BW,), jnp.float32),
        pltpu.SemaphoreType.DMA,
        pltpu.SemaphoreType.DMA,
    ],
)
def _fpmc_sc(seq_idx_hbm, user_hbm, item_hbm, seqlen_hbm,
             wui_hbm, wiu_hbm, wli_hbm, wil_hbm, out_hbm,
             seq_idx_v, user_v, item_v, seqlen_v,
             vui_v, viu_v, vil_v, acc_v, buf_v, out_v, sem0, sem1):
    wid = lax.axis_index("s") * NC + lax.axis_index("c")
    base = wid * BW

    pltpu.sync_copy(seq_idx_hbm.at[wid], seq_idx_v)
    pltpu.sync_copy(user_hbm.at[wid], user_v)
    pltpu.sync_copy(item_hbm.at[wid], item_v)
    pltpu.sync_copy(seqlen_hbm.at[wid], seqlen_v)

    zero = jnp.zeros((16,), jnp.float32)

    def zbody(i, carry):
        acc_v[i, pl.ds(0, 16)] = zero
        acc_v[i, pl.ds(16, 16)] = zero
        return carry
    lax.fori_loop(0, BW, zbody, 0)

    for c in range(CH):
        dst = pl.ds(c * CL, CL)
        pltpu.async_copy(wui_hbm.at[user_v.at[c]], vui_v.at[dst], sem1)
        pltpu.async_copy(wiu_hbm.at[item_v.at[c]], viu_v.at[dst], sem1)
        pltpu.async_copy(wil_hbm.at[item_v.at[c]], vil_v.at[dst], sem1)

    def fire(t, carry):
        l = t // CH
        c = lax.rem(t, CH)
        pltpu.async_copy(wli_hbm.at[seq_idx_v.at[l, c]],
                         acc_v.at[pl.ds(c * CL, CL)], sem0, add=True)
        return carry
    lax.fori_loop(0, L * CH, fire, 0)

    def drain(t, carry):
        pltpu.make_async_copy(wli_hbm.at[pl.ds(0, CL)], buf_v, sem0).wait()
        return carry
    lax.fori_loop(0, L * CH, drain, 0)

    def drain1(t, carry):
        pltpu.make_async_copy(wli_hbm.at[pl.ds(0, CL)], buf_v, sem1).wait()
        return carry
    lax.fori_loop(0, 3 * CH, drain1, 0)

    iota16 = lax.iota(jnp.int32, 16)

    def gbody(g, carry):
        rows = g * 16 + iota16
        a1 = zero
        a2 = zero
        for d in range(D):
            cols = jnp.full((16,), d, jnp.int32)
            vu = plsc.load_gather(vui_v, [rows, cols])
            vi = plsc.load_gather(viu_v, [rows, cols])
            sacc = plsc.load_gather(acc_v, [rows, cols])
            vl = plsc.load_gather(vil_v, [rows, cols])
            a1 = a1 + vu * vi
            a2 = a2 + sacc * vl
        off = pl.multiple_of(g * 16, 16)
        sl = seqlen_v[pl.ds(off, 16)]
        out_v[pl.ds(off, 16)] = a1 + a2 / sl
        return carry
    lax.fori_loop(0, BW // 16, gbody, 0)

    pltpu.sync_copy(out_v, out_hbm.at[pl.ds(base, BW)])


def kernel(user, item, item_seq, seq_len, W_UI, W_IU, W_LI, W_IL):
    user_r = user.astype(jnp.int32).reshape(NW, CH, CL)
    item_r = item.astype(jnp.int32).reshape(NW, CH, CL)
    seq_r = item_seq.astype(jnp.int32).reshape(NW, CH, CL, L).transpose(0, 3, 1, 2)
    sl_r = seq_len.reshape(NW, BW)
    return _fpmc_sc(seq_r, user_r, item_r, sl_r, W_UI, W_IU, W_LI, W_IL)
